# baseline (speedup 1.0000x reference)
import functools

import jax
import jax.numpy as jnp
from jax import lax
from jax.experimental import pallas as pl
from jax.experimental.pallas import tpu as pltpu
from jax.experimental.pallas import tpu_sc as plsc

_B, _C_IN, _N = 8, 256, 196
_D, _K = 64, 1024
_BN = _B * _N
_BNP = 1792


def _split3(x):
    h = x.astype(jnp.bfloat16)
    r = x - h.astype(jnp.float32)
    m = r.astype(jnp.bfloat16)
    l = (r - m.astype(jnp.float32)).astype(jnp.bfloat16)
    return h, m, l


def _idx_body(z_ref, w_ref, emb_ref, idx_ref):
    emb = emb_ref[...]

    wb = w_ref[...].astype(jnp.bfloat16)
    z_all = jnp.concatenate([z_ref[b] for b in range(_B)], axis=1)
    ze = jnp.dot(wb, z_all.astype(jnp.bfloat16),
                 preferred_element_type=jnp.float32)

    eh = emb.astype(jnp.bfloat16)
    el = (emb - eh.astype(jnp.float32)).astype(jnp.bfloat16)
    e_sq = jnp.sum(emb * emb, axis=1, keepdims=True)
    qh, qm, ql = _split3(e_sq)
    zh = ze.astype(jnp.bfloat16)
    zl = (ze - zh.astype(jnp.float32)).astype(jnp.bfloat16)

    lhs = jnp.concatenate([-2.0 * eh, -2.0 * eh, -2.0 * el, qh, qm, ql],
                          axis=1)
    ones = jnp.ones((1, _BN), dtype=jnp.bfloat16)
    rhs = jnp.concatenate([zh, zl, zh, ones, ones, ones], axis=0)
    s = jnp.dot(lhs, rhs, preferred_element_type=jnp.float32)

    m = jnp.min(s, axis=0, keepdims=True)
    kio = jax.lax.broadcasted_iota(jnp.int32, (_K, _BN), 0)
    idx = jnp.min(jnp.where(s <= m, kio, _K), axis=0)
    idx_ref[...] = jnp.concatenate(
        [idx, jnp.zeros((_BNP - _BN,), jnp.int32)])


def _indices(z, W, emb):
    return pl.pallas_call(
        _idx_body,
        in_specs=[
            pl.BlockSpec(memory_space=pltpu.VMEM),
            pl.BlockSpec(memory_space=pltpu.VMEM),
            pl.BlockSpec(memory_space=pltpu.VMEM),
        ],
        out_specs=pl.BlockSpec(memory_space=pltpu.VMEM),
        out_shape=jax.ShapeDtypeStruct((_BNP,), jnp.int32),
    )(z, W, emb)


_NC, _NS = 2, 16
_NW = _NC * _NS
_BPW = _BNP // _NW


@functools.partial(
    pl.kernel,
    mesh=plsc.VectorSubcoreMesh(core_axis_name="c", subcore_axis_name="s"),
    compiler_params=pltpu.CompilerParams(use_tc_tiling_on_sc=False),
    out_type=jax.ShapeDtypeStruct((_BNP, _D), jnp.float32),
    scratch_types=[
        pltpu.VMEM((_BPW,), jnp.int32),
        pltpu.VMEM((_BPW, _D), jnp.float32),
        pltpu.SemaphoreType.DMA,
    ],
)
def _sc_gather(emb_hbm, idx_hbm, out_hbm, idx_v, rows_v, sem):
    wid = lax.axis_index("s") * _NC + lax.axis_index("c")
    base = wid * _BPW
    pltpu.sync_copy(idx_hbm.at[pl.ds(base, _BPW)], idx_v)
    pltpu.async_copy(emb_hbm.at[idx_v], rows_v, sem).wait()
    pltpu.sync_copy(rows_v, out_hbm.at[pl.ds(base, _BPW)])


def kernel(z, W, emb):
    idx = _indices(z, W, emb)
    rows = _sc_gather(emb, idx)
    return rows[:_BN].reshape(_B, _N, _D).transpose(0, 2, 1)


# speedup vs baseline: 3.1140x; 3.1140x over previous
import jax
import jax.numpy as jnp
from jax.experimental import pallas as pl
from jax.experimental.pallas import tpu as pltpu

_B, _C_IN, _N = 8, 256, 196
_D, _K = 64, 1024
_BN = _B * _N


def _split3(x):
    h = x.astype(jnp.bfloat16)
    r = x - h.astype(jnp.float32)
    m = r.astype(jnp.bfloat16)
    l = (r - m.astype(jnp.float32)).astype(jnp.bfloat16)
    return h, m, l


def _vq_body(z_ref, w_ref, emb_ref, out_ref):
    emb = emb_ref[...]

    wb = w_ref[...].astype(jnp.bfloat16)
    ze = jnp.concatenate(
        [jnp.dot(wb, z_ref[b].astype(jnp.bfloat16),
                 preferred_element_type=jnp.float32) for b in range(_B)],
        axis=1)

    eh = emb.astype(jnp.bfloat16)
    el = (emb - eh.astype(jnp.float32)).astype(jnp.bfloat16)
    e_sq = jnp.sum(emb * emb, axis=1, keepdims=True)
    qh, qm, ql = _split3(e_sq)
    zh = ze.astype(jnp.bfloat16)
    zl = (ze - zh.astype(jnp.float32)).astype(jnp.bfloat16)

    lhs = jnp.concatenate([-2.0 * eh, -2.0 * eh, -2.0 * el, qh, qm, ql],
                          axis=1)
    ones = jnp.ones((1, _BN), dtype=jnp.bfloat16)
    rhs = jnp.concatenate([zh, zl, zh, ones, ones, ones], axis=0)
    s = jnp.dot(lhs, rhs, preferred_element_type=jnp.float32)

    m = jnp.min(s, axis=0, keepdims=True)
    kio = jax.lax.broadcasted_iota(jnp.int32, (_K, _BN), 0)
    idx = jnp.min(jnp.where(s <= m, kio, _K), axis=0)
    onehot = (kio == idx[None, :]).astype(jnp.bfloat16)
    zq2 = jnp.dot(jnp.concatenate([eh.T, el.T], axis=0), onehot,
                  preferred_element_type=jnp.float32)
    zq = zq2[:_D] + zq2[_D:]
    for b in range(_B):
        out_ref[b] = zq[:, b * _N:(b + 1) * _N]


def kernel(z, W, emb):
    return pl.pallas_call(
        _vq_body,
        in_specs=[
            pl.BlockSpec(memory_space=pltpu.VMEM),
            pl.BlockSpec(memory_space=pltpu.VMEM),
            pl.BlockSpec(memory_space=pltpu.VMEM),
        ],
        out_specs=pl.BlockSpec(memory_space=pltpu.VMEM),
        out_shape=jax.ShapeDtypeStruct((_B, _D, _N), jnp.float32),
    )(z, W, emb)


# speedup vs baseline: 3.1535x; 1.0127x over previous
import jax
import jax.numpy as jnp
from jax.experimental import pallas as pl
from jax.experimental.pallas import tpu as pltpu

_B, _C_IN, _N = 8, 256, 196
_D, _K = 64, 1024
_BN = _B * _N


def _split3(x):
    h = x.astype(jnp.bfloat16)
    r = x - h.astype(jnp.float32)
    m = r.astype(jnp.bfloat16)
    l = (r - m.astype(jnp.float32)).astype(jnp.bfloat16)
    return h, m, l


def _vq_body(z_ref, w_ref, emb_ref, out_ref):
    emb = emb_ref[...]

    wb = w_ref[...].astype(jnp.bfloat16)
    z_all = jnp.concatenate([z_ref[b] for b in range(_B)], axis=1)
    ze = jnp.dot(wb, z_all.astype(jnp.bfloat16),
                 preferred_element_type=jnp.float32)

    eh = emb.astype(jnp.bfloat16)
    el = (emb - eh.astype(jnp.float32)).astype(jnp.bfloat16)
    e_sq = jnp.sum(emb * emb, axis=1, keepdims=True)
    qh, qm, ql = _split3(e_sq)
    zh = ze.astype(jnp.bfloat16)
    zl = (ze - zh.astype(jnp.float32)).astype(jnp.bfloat16)

    lhs = jnp.concatenate([-2.0 * eh, -2.0 * eh, -2.0 * el, qh, qm, ql],
                          axis=1)
    ones = jnp.ones((1, _BN), dtype=jnp.bfloat16)
    rhs = jnp.concatenate([zh, zl, zh, ones, ones, ones], axis=0)
    s = jnp.dot(lhs, rhs, preferred_element_type=jnp.float32)

    m = jnp.min(s, axis=0, keepdims=True)
    kio = jax.lax.broadcasted_iota(jnp.int32, (_K, _BN), 0)
    idx = jnp.min(jnp.where(s <= m, kio, _K), axis=0)
    onehot = (kio == idx[None, :]).astype(jnp.bfloat16)
    zq2 = jnp.dot(jnp.concatenate([eh.T, el.T], axis=0), onehot,
                  preferred_element_type=jnp.float32)
    zq = zq2[:_D] + zq2[_D:]
    for b in range(_B):
        out_ref[b] = zq[:, b * _N:(b + 1) * _N]


def kernel(z, W, emb):
    return pl.pallas_call(
        _vq_body,
        in_specs=[
            pl.BlockSpec(memory_space=pltpu.VMEM),
            pl.BlockSpec(memory_space=pltpu.VMEM),
            pl.BlockSpec(memory_space=pltpu.VMEM),
        ],
        out_specs=pl.BlockSpec(memory_space=pltpu.VMEM),
        out_shape=jax.ShapeDtypeStruct((_B, _D, _N), jnp.float32),
    )(z, W, emb)
